# baseline (speedup 1.0000x reference)
import functools

import jax
import jax.numpy as jnp
from jax import lax
from jax.experimental import pallas as pl
from jax.experimental.pallas import tpu as pltpu
from jax.experimental.pallas import tpu_sc as plsc


def _route_block(x_ref, gw_ref, gb_ref, xa_ref, eidx_ref, rank_ref, cnts_ref,
                 cnt_scr):
    c = pl.program_id(0)
    x = x_ref[...]
    logits = jnp.dot(x, gw_ref[...].T,
                     preferred_element_type=jnp.float32) + gb_ref[...]
    m = jnp.max(logits, axis=-1, keepdims=True)
    top = 1.0 / jnp.sum(jnp.exp(logits - m), axis=-1, keepdims=True)
    E = logits.shape[-1]
    ecol = lax.broadcasted_iota(jnp.int32, logits.shape, 1)
    amax = jnp.min(jnp.where(logits == m, ecol, E), axis=-1, keepdims=True)
    scale = E * top

    Tg, H = x.shape
    HW = (H + 256) // 2
    va = scale * x
    lane = lax.broadcasted_iota(jnp.int32, (Tg, 2 * HW - H), 1)
    scol = jnp.where(lane == 0, scale, 0.0)
    lo = va[:, :HW]
    hi = jnp.concatenate([va[:, HW:], scol], axis=1)

    ulo = lax.bitcast_convert_type(lo, jnp.uint32) + jnp.uint32(0x8000)
    uhi = lax.bitcast_convert_type(hi, jnp.uint32) + jnp.uint32(0x8000)
    word = (ulo >> 16) | (uhi & jnp.uint32(0xFFFF0000))
    xa_ref[...] = lax.bitcast_convert_type(word, jnp.int32)
    eidx_ref[...] = amax[:, 0]

    oh = (amax == ecol).astype(jnp.float32)
    ri = lax.broadcasted_iota(jnp.int32, (Tg, Tg), 0)
    ci = lax.broadcasted_iota(jnp.int32, (Tg, Tg), 1)
    tri = (ri >= ci).astype(jnp.float32)
    cum = jnp.dot(tri, oh, preferred_element_type=jnp.float32)

    @pl.when(c == 0)
    def _zero():
        cnt_scr[...] = jnp.zeros_like(cnt_scr)

    carry = cnt_scr[0, :]
    rank_in = jnp.sum(oh * cum, axis=1) - 1.0
    carrytok = jnp.sum(oh * carry[None, :], axis=1)
    rank_ref[...] = (rank_in + carrytok).astype(jnp.int32)
    totals = carry + jnp.sum(oh, axis=0)
    cnt_scr[0, :] = totals
    cnts_ref[...] = totals[None, :]


def _plan_block(cnts_ref, eidx_ref, rank_ref, sched_ref, slot_ref, *, Tc, G):
    c = pl.program_id(0)
    E = cnts_ref.shape[1]
    counts = cnts_ref[0, :]
    ei = lax.broadcasted_iota(jnp.int32, (E, E), 0)
    ej = lax.broadcasted_iota(jnp.int32, (E, E), 1)
    ut_strict = (ei < ej).astype(jnp.float32)
    ut_incl = (ei <= ej).astype(jnp.float32)
    off = counts @ ut_strict
    inv = jnp.float32(1.0 / Tc)
    c_lo = jnp.floor(off * inv)
    c_hi = jnp.floor((off + counts - 1.0) * inv)
    span = jnp.where(counts > 0.0, c_hi - c_lo + 1.0, 0.0)
    incl = span @ ut_incl
    basex = incl - span
    lane_last = lax.broadcasted_iota(jnp.int32, (E,), 0) == (E - 1)
    g_act = jnp.sum(jnp.where(lane_last, incl, 0.0))

    @pl.when(c == 0)
    def _sched():
        gi = lax.broadcasted_iota(jnp.int32, (G, 1), 0).astype(jnp.float32)
        e_raw = jnp.minimum(
            jnp.sum((gi >= incl[None, :]).astype(jnp.float32), axis=1),
            jnp.float32(E - 1))
        oh = (e_raw[:, None] ==
              lax.broadcasted_iota(jnp.int32, (G, E), 1).astype(jnp.float32)
              ).astype(jnp.float32)
        take = lambda v: jnp.sum(oh * v[None, :], axis=1)
        c_raw = take(c_lo) + (gi[:, 0] - take(basex))
        st_raw = jnp.maximum(take(off), c_raw * Tc)
        en_raw = jnp.minimum(take(off + counts), (c_raw + 1.0) * Tc)
        act = gi[:, 0] < g_act
        lmask = (gi[:, 0] == g_act - 1.0).astype(jnp.float32)
        e_last = jnp.sum(e_raw * lmask)
        c_last = jnp.sum(c_raw * lmask)
        sched_ref[0, :] = jnp.where(act, e_raw, e_last).astype(jnp.int32)
        sched_ref[1, :] = jnp.where(act, c_raw, c_last).astype(jnp.int32)
        sched_ref[2, :] = jnp.where(act, st_raw, 0.0).astype(jnp.int32)
        sched_ref[3, :] = jnp.where(act, en_raw, 0.0).astype(jnp.int32)

    eidx = eidx_ref[...]
    T2 = eidx.shape[0]
    ohe = (eidx[:, None] ==
           lax.broadcasted_iota(jnp.int32, (T2, E), 1)).astype(jnp.float32)
    base = jnp.sum(ohe * off[None, :], axis=1)
    slot_ref[...] = rank_ref[...] + base.astype(jnp.int32)


def _gmm_block(s_ref, xs_ref, ew_ref, eb_ref, o_ref, w16_scr):
    g = pl.program_id(0)
    c = s_ref[1, g]
    st = s_ref[2, g]
    en = s_ref[3, g]
    prev_c = s_ref[1, jnp.maximum(g - 1, 0)]
    newc = jnp.logical_or(g == 0, c != prev_c)
    e_now = s_ref[0, g]
    e_prev = s_ref[0, jnp.maximum(g - 1, 0)]
    newe = jnp.logical_or(g == 0, e_now != e_prev)
    Tc = o_ref.shape[0]
    H = o_ref.shape[1]

    @pl.when(newe)
    def _conv():
        w16_scr[...] = ew_ref[0].astype(jnp.bfloat16)

    @pl.when(st < en)
    def _active():
        HW = xs_ref.shape[1]
        xw = lax.bitcast_convert_type(xs_ref[...], jnp.uint32)
        lo = lax.bitcast_convert_type(xw << 16, jnp.float32)
        hi = lax.bitcast_convert_type(xw & jnp.uint32(0xFFFF0000), jnp.float32)
        w16 = w16_scr[...]
        y = jnp.dot(lo.astype(jnp.bfloat16), w16[:, :HW].T,
                    preferred_element_type=jnp.float32)
        y = y + jnp.dot(hi[:, :H - HW].astype(jnp.bfloat16), w16[:, HW:].T,
                        preferred_element_type=jnp.float32)
        gcol = jnp.sum(hi[:, H - HW:], axis=1, keepdims=True)
        rows = c * Tc + lax.broadcasted_iota(jnp.int32, (Tc, 1), 0)
        msk = ((rows >= st) & (rows < en)).astype(jnp.float32)
        contrib = msk * (y + gcol * eb_ref[0])

        @pl.when(newc)
        def _init():
            o_ref[...] = contrib

        @pl.when(jnp.logical_not(newc))
        def _acc():
            o_ref[...] = o_ref[...] + contrib


def kernel(x, gate_w, gate_b, expert_w, expert_b):
    B, S, H = x.shape
    E = gate_w.shape[0]
    N = B * S
    x2 = x.reshape(N, H)
    HW = (H + 256) // 2
    Tg = min(512, N)
    Cg = N // Tg
    xa, eidx, rank, cnts = pl.pallas_call(
        _route_block,
        grid=(Cg,),
        in_specs=[
            pl.BlockSpec((Tg, H), lambda c: (c, 0)),
            pl.BlockSpec((E, H), lambda c: (0, 0)),
            pl.BlockSpec((E,), lambda c: (0,)),
        ],
        out_specs=[
            pl.BlockSpec((Tg, HW), lambda c: (c, 0)),
            pl.BlockSpec((Tg,), lambda c: (c,)),
            pl.BlockSpec((Tg,), lambda c: (c,)),
            pl.BlockSpec((1, E), lambda c: (0, 0)),
        ],
        out_shape=[
            jax.ShapeDtypeStruct((N, HW), jnp.int32),
            jax.ShapeDtypeStruct((N,), jnp.int32),
            jax.ShapeDtypeStruct((N,), jnp.int32),
            jax.ShapeDtypeStruct((1, E), jnp.float32),
        ],
        scratch_shapes=[pltpu.VMEM((1, E), jnp.float32)],
    )(x2, gate_w, gate_b)

    Tc = min(256, N)
    C = N // Tc
    G = C + E - 1
    Tg2 = Tg
    sched, slot = pl.pallas_call(
        functools.partial(_plan_block, Tc=Tc, G=G),
        grid=(N // Tg2,),
        in_specs=[
            pl.BlockSpec((1, E), lambda c: (0, 0)),
            pl.BlockSpec((Tg2,), lambda c: (c,)),
            pl.BlockSpec((Tg2,), lambda c: (c,)),
        ],
        out_specs=[
            pl.BlockSpec((4, G), lambda c: (0, 0)),
            pl.BlockSpec((Tg2,), lambda c: (c,)),
        ],
        out_shape=[
            jax.ShapeDtypeStruct((4, G), jnp.int32),
            jax.ShapeDtypeStruct((N,), jnp.int32),
        ],
    )(cnts, eidx, rank)

    NC, NS = 2, 16
    NW = NC * NS
    PW = N // NW
    BSZ = min(64, PW)
    K = PW // BSZ
    BSZC = min(32, PW)
    KC = PW // BSZC
    mesh = plsc.VectorSubcoreMesh(core_axis_name="c", subcore_axis_name="s",
                                  num_cores=NC, num_subcores=NS)

    @functools.partial(
        pl.kernel, mesh=mesh,
        out_type=jax.ShapeDtypeStruct((N, HW), jnp.int32),
        scratch_types=[
            pltpu.VMEM((BSZ,), jnp.int32),
            pltpu.VMEM((BSZ,), jnp.int32),
            pltpu.VMEM((BSZ, HW), jnp.int32),
            pltpu.VMEM((BSZ, HW), jnp.int32),
            pltpu.SemaphoreType.DMA,
            pltpu.SemaphoreType.DMA,
            pltpu.SemaphoreType.DMA,
            pltpu.SemaphoreType.DMA,
            pltpu.SemaphoreType.DMA,
            pltpu.SemaphoreType.DMA,
        ],
    )
    def _dispatch(xa_hbm, slot_hbm, xs_hbm, ix0, ix1, rw0, rw1,
                  si0, si1, sr0, sr1, ss0, ss1):
        wid = lax.axis_index("s") * NC + lax.axis_index("c")
        ixb, rwb = (ix0, ix1), (rw0, rw1)
        sib, srb, ssb = (si0, si1), (sr0, sr1), (ss0, ss1)

        def r0(j):
            return pl.multiple_of(wid * PW + j * BSZ, BSZ)

        loads = {0: (pltpu.async_copy(slot_hbm.at[pl.ds(r0(0), BSZ)], ixb[0], sib[0]),
                     pltpu.async_copy(xa_hbm.at[pl.ds(r0(0), BSZ)], rwb[0], srb[0]))}
        scat = {}
        for j in range(K):
            b = j & 1
            loads[j][0].wait()
            loads[j][1].wait()
            scat[j] = pltpu.async_copy(rwb[b], xs_hbm.at[ixb[b]], ssb[b])
            if j + 1 < K:
                nb = 1 - b
                if j >= 1:
                    scat[j - 1].wait()
                loads[j + 1] = (
                    pltpu.async_copy(slot_hbm.at[pl.ds(r0(j + 1), BSZ)], ixb[nb], sib[nb]),
                    pltpu.async_copy(xa_hbm.at[pl.ds(r0(j + 1), BSZ)], rwb[nb], srb[nb]))
        if K >= 2:
            scat[K - 2].wait()
        scat[K - 1].wait()

    xs = _dispatch(xa, slot)

    ys = pl.pallas_call(
        _gmm_block,
        grid_spec=pltpu.PrefetchScalarGridSpec(
            num_scalar_prefetch=1,
            grid=(G,),
            in_specs=[
                pl.BlockSpec((Tc, HW), lambda g, s: (s[1, g], 0)),
                pl.BlockSpec((1, H, H), lambda g, s: (s[0, g], 0, 0)),
                pl.BlockSpec((1, 1, H), lambda g, s: (s[0, g], 0, 0)),
            ],
            out_specs=pl.BlockSpec((Tc, H), lambda g, s: (s[1, g], 0)),
            scratch_shapes=[pltpu.VMEM((H, H), jnp.bfloat16)],
        ),
        out_shape=jax.ShapeDtypeStruct((N, H), jnp.float32),
    )(sched, xs, expert_w, expert_b.reshape(E, 1, H))

    @functools.partial(
        pl.kernel, mesh=mesh,
        out_type=jax.ShapeDtypeStruct((N, H), jnp.float32),
        scratch_types=[
            pltpu.VMEM((BSZC,), jnp.int32),
            pltpu.VMEM((BSZC,), jnp.int32),
            pltpu.VMEM((BSZC, H), jnp.float32),
            pltpu.VMEM((BSZC, H), jnp.float32),
            pltpu.SemaphoreType.DMA,
            pltpu.SemaphoreType.DMA,
            pltpu.SemaphoreType.DMA,
            pltpu.SemaphoreType.DMA,
            pltpu.SemaphoreType.DMA,
            pltpu.SemaphoreType.DMA,
        ],
    )
    def _combine(ys_hbm, slot_hbm, out_hbm, ix0, ix1, rw0, rw1,
                 si0, si1, sg0, sg1, ss0, ss1):
        wid = lax.axis_index("s") * NC + lax.axis_index("c")
        ixb, rwb = (ix0, ix1), (rw0, rw1)
        sib, sgb, ssb = (si0, si1), (sg0, sg1), (ss0, ss1)

        def r0(j):
            return pl.multiple_of(wid * PW + j * BSZC, BSZC)

        idxl = {0: pltpu.async_copy(slot_hbm.at[pl.ds(r0(0), BSZC)], ixb[0], sib[0])}
        sto = {}
        for j in range(KC):
            b = j & 1
            idxl[j].wait()
            if j >= 2:
                sto[j - 2].wait()
            gat = pltpu.async_copy(ys_hbm.at[ixb[b]], rwb[b], sgb[b])
            if j + 1 < KC:
                idxl[j + 1] = pltpu.async_copy(
                    slot_hbm.at[pl.ds(r0(j + 1), BSZC)], ixb[1 - b], sib[1 - b])
            gat.wait()
            sto[j] = pltpu.async_copy(rwb[b], out_hbm.at[pl.ds(r0(j), BSZC)], ssb[b])
        if KC >= 2:
            sto[KC - 2].wait()
        sto[KC - 1].wait()

    out = _combine(ys, slot)
    return out.reshape(B, S, H)


# speedup vs baseline: 1.0367x; 1.0367x over previous
import functools

import jax
import jax.numpy as jnp
from jax import lax
from jax.experimental import pallas as pl
from jax.experimental.pallas import tpu as pltpu
from jax.experimental.pallas import tpu_sc as plsc


def _route_block(x_ref, gw_ref, gb_ref, xa_ref, eidx_ref, rank_ref, cnts_ref,
                 cnt_scr):
    c = pl.program_id(0)
    x = x_ref[...]
    logits = jnp.dot(x, gw_ref[...].T,
                     preferred_element_type=jnp.float32) + gb_ref[...]
    m = jnp.max(logits, axis=-1, keepdims=True)
    top = 1.0 / jnp.sum(jnp.exp(logits - m), axis=-1, keepdims=True)
    E = logits.shape[-1]
    ecol = lax.broadcasted_iota(jnp.int32, logits.shape, 1)
    amax = jnp.min(jnp.where(logits == m, ecol, E), axis=-1, keepdims=True)
    scale = E * top

    Tg, H = x.shape
    HW = (H + 256) // 2
    va = scale * x
    lane = lax.broadcasted_iota(jnp.int32, (Tg, 2 * HW - H), 1)
    scol = jnp.where(lane == 0, scale, 0.0)
    lo = va[:, :HW]
    hi = jnp.concatenate([va[:, HW:], scol], axis=1)

    ulo = lax.bitcast_convert_type(lo, jnp.uint32) + jnp.uint32(0x8000)
    uhi = lax.bitcast_convert_type(hi, jnp.uint32) + jnp.uint32(0x8000)
    word = (ulo >> 16) | (uhi & jnp.uint32(0xFFFF0000))
    xa_ref[...] = lax.bitcast_convert_type(word, jnp.int32)
    eidx_ref[...] = amax[:, 0]

    oh = (amax == ecol).astype(jnp.float32)
    ri = lax.broadcasted_iota(jnp.int32, (Tg, Tg), 0)
    ci = lax.broadcasted_iota(jnp.int32, (Tg, Tg), 1)
    tri = (ri >= ci).astype(jnp.float32)
    cum = jnp.dot(tri, oh, preferred_element_type=jnp.float32)

    @pl.when(c == 0)
    def _zero():
        cnt_scr[...] = jnp.zeros_like(cnt_scr)

    carry = cnt_scr[0, :]
    rank_in = jnp.sum(oh * cum, axis=1) - 1.0
    carrytok = jnp.sum(oh * carry[None, :], axis=1)
    rank_ref[...] = (rank_in + carrytok).astype(jnp.int32)
    totals = carry + jnp.sum(oh, axis=0)
    cnt_scr[0, :] = totals
    cnts_ref[...] = totals[None, :]


def _gmm_block(s_ref, xs_ref, ew_ref, eb_ref, o_ref, w16_scr):
    g = pl.program_id(0)
    c = s_ref[1, g]
    st = s_ref[2, g]
    en = s_ref[3, g]
    prev_c = s_ref[1, jnp.maximum(g - 1, 0)]
    newc = jnp.logical_or(g == 0, c != prev_c)
    e_now = s_ref[0, g]
    e_prev = s_ref[0, jnp.maximum(g - 1, 0)]
    newe = jnp.logical_or(g == 0, e_now != e_prev)
    Tc = o_ref.shape[0]
    H = o_ref.shape[1]

    @pl.when(newe)
    def _conv():
        w16_scr[...] = ew_ref[0].astype(jnp.bfloat16)

    @pl.when(st < en)
    def _active():
        HW = xs_ref.shape[1]
        xw = lax.bitcast_convert_type(xs_ref[...], jnp.uint32)
        lo = lax.bitcast_convert_type(xw << 16, jnp.float32)
        hi = lax.bitcast_convert_type(xw & jnp.uint32(0xFFFF0000), jnp.float32)
        w16 = w16_scr[...]
        y = jnp.dot(lo.astype(jnp.bfloat16), w16[:, :HW].T,
                    preferred_element_type=jnp.float32)
        y = y + jnp.dot(hi[:, :H - HW].astype(jnp.bfloat16), w16[:, HW:].T,
                        preferred_element_type=jnp.float32)
        gcol = jnp.sum(hi[:, H - HW:], axis=1, keepdims=True)
        rows = c * Tc + lax.broadcasted_iota(jnp.int32, (Tc, 1), 0)
        msk = ((rows >= st) & (rows < en)).astype(jnp.float32)
        contrib = msk * (y + gcol * eb_ref[0])

        @pl.when(newc)
        def _init():
            o_ref[...] = contrib

        @pl.when(jnp.logical_not(newc))
        def _acc():
            o_ref[...] = o_ref[...] + contrib


def kernel(x, gate_w, gate_b, expert_w, expert_b):
    B, S, H = x.shape
    E = gate_w.shape[0]
    N = B * S
    x2 = x.reshape(N, H)
    HW = (H + 256) // 2
    Tg = min(512, N)
    Cg = N // Tg
    xa, eidx, rank, cnts = pl.pallas_call(
        _route_block,
        grid=(Cg,),
        in_specs=[
            pl.BlockSpec((Tg, H), lambda c: (c, 0)),
            pl.BlockSpec((E, H), lambda c: (0, 0)),
            pl.BlockSpec((E,), lambda c: (0,)),
        ],
        out_specs=[
            pl.BlockSpec((Tg, HW), lambda c: (c, 0)),
            pl.BlockSpec((Tg,), lambda c: (c,)),
            pl.BlockSpec((Tg,), lambda c: (c,)),
            pl.BlockSpec((1, E), lambda c: (0, 0)),
        ],
        out_shape=[
            jax.ShapeDtypeStruct((N, HW), jnp.int32),
            jax.ShapeDtypeStruct((N,), jnp.int32),
            jax.ShapeDtypeStruct((N,), jnp.int32),
            jax.ShapeDtypeStruct((1, E), jnp.float32),
        ],
        scratch_shapes=[pltpu.VMEM((1, E), jnp.float32)],
    )(x2, gate_w, gate_b)

    counts = cnts[0].astype(jnp.int32)
    off = jnp.concatenate(
        [jnp.zeros((1,), jnp.int32), jnp.cumsum(counts)[:-1].astype(jnp.int32)])
    er = jnp.arange(E, dtype=jnp.int32)
    slot = rank + jnp.sum(
        jnp.where(eidx[:, None] == er[None, :], off[None, :], 0), axis=1)

    Tc = min(256, N)
    C = N // Tc
    G = C + E - 1
    c_lo = off // Tc
    c_hi = (off + counts - 1) // Tc
    span = jnp.where(counts > 0, c_hi - c_lo + 1, 0)
    incl = jnp.cumsum(span)
    basex = incl - span
    g_act = incl[-1]
    gi = jnp.arange(G, dtype=jnp.int32)
    e_raw = jnp.minimum(
        jnp.sum((gi[:, None] >= incl[None, :]).astype(jnp.int32), axis=1), E - 1)
    c_raw = jnp.take(c_lo, e_raw) + (gi - jnp.take(basex, e_raw))
    st_raw = jnp.maximum(jnp.take(off, e_raw), c_raw * Tc)
    en_raw = jnp.minimum(jnp.take(off + counts, e_raw), (c_raw + 1) * Tc)
    act = gi < g_act
    e_last = jnp.take(e_raw, g_act - 1)
    c_last = jnp.take(c_raw, g_act - 1)
    sched = jnp.stack([
        jnp.where(act, e_raw, e_last),
        jnp.where(act, c_raw, c_last),
        jnp.where(act, st_raw, 0),
        jnp.where(act, en_raw, 0),
    ]).astype(jnp.int32)

    NC, NS = 2, 16
    NW = NC * NS
    PW = N // NW
    BSZ = min(64, PW)
    K = PW // BSZ
    BSZC = min(32, PW)
    KC = PW // BSZC
    mesh = plsc.VectorSubcoreMesh(core_axis_name="c", subcore_axis_name="s",
                                  num_cores=NC, num_subcores=NS)

    @functools.partial(
        pl.kernel, mesh=mesh,
        out_type=jax.ShapeDtypeStruct((N, HW), jnp.int32),
        scratch_types=[
            pltpu.VMEM((BSZ,), jnp.int32),
            pltpu.VMEM((BSZ,), jnp.int32),
            pltpu.VMEM((BSZ, HW), jnp.int32),
            pltpu.VMEM((BSZ, HW), jnp.int32),
            pltpu.SemaphoreType.DMA,
            pltpu.SemaphoreType.DMA,
            pltpu.SemaphoreType.DMA,
            pltpu.SemaphoreType.DMA,
            pltpu.SemaphoreType.DMA,
            pltpu.SemaphoreType.DMA,
        ],
    )
    def _dispatch(xa_hbm, slot_hbm, xs_hbm, ix0, ix1, rw0, rw1,
                  si0, si1, sr0, sr1, ss0, ss1):
        wid = lax.axis_index("s") * NC + lax.axis_index("c")
        ixb, rwb = (ix0, ix1), (rw0, rw1)
        sib, srb, ssb = (si0, si1), (sr0, sr1), (ss0, ss1)

        def r0(j):
            return pl.multiple_of(wid * PW + j * BSZ, BSZ)

        loads = {0: (pltpu.async_copy(slot_hbm.at[pl.ds(r0(0), BSZ)], ixb[0], sib[0]),
                     pltpu.async_copy(xa_hbm.at[pl.ds(r0(0), BSZ)], rwb[0], srb[0]))}
        scat = {}
        for j in range(K):
            b = j & 1
            loads[j][0].wait()
            loads[j][1].wait()
            scat[j] = pltpu.async_copy(rwb[b], xs_hbm.at[ixb[b]], ssb[b])
            if j + 1 < K:
                nb = 1 - b
                if j >= 1:
                    scat[j - 1].wait()
                loads[j + 1] = (
                    pltpu.async_copy(slot_hbm.at[pl.ds(r0(j + 1), BSZ)], ixb[nb], sib[nb]),
                    pltpu.async_copy(xa_hbm.at[pl.ds(r0(j + 1), BSZ)], rwb[nb], srb[nb]))
        if K >= 2:
            scat[K - 2].wait()
        scat[K - 1].wait()

    xs = _dispatch(xa, slot)

    ys = pl.pallas_call(
        _gmm_block,
        grid_spec=pltpu.PrefetchScalarGridSpec(
            num_scalar_prefetch=1,
            grid=(G,),
            in_specs=[
                pl.BlockSpec((Tc, HW), lambda g, s: (s[1, g], 0)),
                pl.BlockSpec((1, H, H), lambda g, s: (s[0, g], 0, 0)),
                pl.BlockSpec((1, 1, H), lambda g, s: (s[0, g], 0, 0)),
            ],
            out_specs=pl.BlockSpec((Tc, H), lambda g, s: (s[1, g], 0)),
            scratch_shapes=[pltpu.VMEM((H, H), jnp.bfloat16)],
        ),
        out_shape=jax.ShapeDtypeStruct((N, H), jnp.float32),
    )(sched, xs, expert_w, expert_b.reshape(E, 1, H))

    @functools.partial(
        pl.kernel, mesh=mesh,
        out_type=jax.ShapeDtypeStruct((N, H), jnp.float32),
        scratch_types=[
            pltpu.VMEM((BSZC,), jnp.int32),
            pltpu.VMEM((BSZC,), jnp.int32),
            pltpu.VMEM((BSZC, H), jnp.float32),
            pltpu.VMEM((BSZC, H), jnp.float32),
            pltpu.SemaphoreType.DMA,
            pltpu.SemaphoreType.DMA,
            pltpu.SemaphoreType.DMA,
            pltpu.SemaphoreType.DMA,
            pltpu.SemaphoreType.DMA,
            pltpu.SemaphoreType.DMA,
        ],
    )
    def _combine(ys_hbm, slot_hbm, out_hbm, ix0, ix1, rw0, rw1,
                 si0, si1, sg0, sg1, ss0, ss1):
        wid = lax.axis_index("s") * NC + lax.axis_index("c")
        ixb, rwb = (ix0, ix1), (rw0, rw1)
        sib, sgb, ssb = (si0, si1), (sg0, sg1), (ss0, ss1)

        def r0(j):
            return pl.multiple_of(wid * PW + j * BSZC, BSZC)

        idxl = {0: pltpu.async_copy(slot_hbm.at[pl.ds(r0(0), BSZC)], ixb[0], sib[0])}
        sto = {}
        for j in range(KC):
            b = j & 1
            idxl[j].wait()
            if j >= 2:
                sto[j - 2].wait()
            gat = pltpu.async_copy(ys_hbm.at[ixb[b]], rwb[b], sgb[b])
            if j + 1 < KC:
                idxl[j + 1] = pltpu.async_copy(
                    slot_hbm.at[pl.ds(r0(j + 1), BSZC)], ixb[1 - b], sib[1 - b])
            gat.wait()
            sto[j] = pltpu.async_copy(rwb[b], out_hbm.at[pl.ds(r0(j), BSZC)], ssb[b])
        if KC >= 2:
            sto[KC - 2].wait()
        sto[KC - 1].wait()

    out = _combine(ys, slot)
    return out.reshape(B, S, H)
